# baseline (speedup 1.0000x reference)
import jax
import jax.numpy as jnp
from jax import lax
from jax.experimental import pallas as pl
from jax.experimental.pallas import tpu as pltpu
from jax.experimental.pallas import tpu_sc as plsc

VOCAB = 1000000
EMB_DIM = 64
BATCH = 16384

_NC = 2
_NS = 16
_LANES = 16
_NW = _NC * _NS
_BPW = BATCH // _NW
_ICHUNK = 128
_NCHUNK = _BPW // _ICHUNK


def _emb_mul_kernel(idx_hbm, repr_hbm, table_hbm, out_hbm,
                    idx_v, rows_v, rep_v, gsem, rsem):
    wid = lax.axis_index("s") * _NC + lax.axis_index("c")
    base = wid * _BPW

    pltpu.sync_copy(idx_hbm.at[wid], idx_v)

    rep_cp = pltpu.async_copy(repr_hbm.at[pl.ds(base, _BPW)], rep_v, rsem)

    gathers = []
    for j in range(_NCHUNK):
        gathers.append(pltpu.async_copy(
            table_hbm.at[idx_v.at[j]],
            rows_v.at[pl.ds(j * _ICHUNK, _ICHUNK)],
            gsem))
    for cp in gathers:
        cp.wait()
    rep_cp.wait()

    def body(i, carry):
        for c in range(EMB_DIM // _LANES):
            sl = pl.ds(c * _LANES, _LANES)
            rows_v[i, sl] = rows_v[i, sl] * rep_v[i, sl]
        return carry

    lax.fori_loop(0, _BPW, body, 0, unroll=4)

    pltpu.sync_copy(rows_v, out_hbm.at[pl.ds(base, _BPW)])


@jax.jit
def kernel(_next_types, representation, table):
    idx = _next_types.reshape(_NW, _NCHUNK, _ICHUNK).astype(jnp.int32)
    rep = representation.reshape(BATCH, EMB_DIM)
    table2 = table.reshape(-1).reshape(VOCAB, EMB_DIM)

    mesh = plsc.VectorSubcoreMesh(core_axis_name="c", subcore_axis_name="s")
    out = pl.kernel(
        _emb_mul_kernel,
        out_type=jax.ShapeDtypeStruct((BATCH, EMB_DIM), jnp.float32),
        mesh=mesh,
        compiler_params=pltpu.CompilerParams(use_tc_tiling_on_sc=False),
        scratch_types=[
            pltpu.VMEM((_NCHUNK, _ICHUNK), jnp.int32),
            pltpu.VMEM((_BPW, EMB_DIM), jnp.float32),
            pltpu.VMEM((_BPW, EMB_DIM), jnp.float32),
            pltpu.SemaphoreType.DMA,
            pltpu.SemaphoreType.DMA,
        ],
    )(idx, rep, table2)
    return out.reshape(BATCH, 1, EMB_DIM)


# speedup vs baseline: 3.0308x; 3.0308x over previous
import jax
import jax.numpy as jnp
from jax import lax
from jax.experimental import pallas as pl
from jax.experimental.pallas import tpu as pltpu
from jax.experimental.pallas import tpu_sc as plsc

VOCAB = 1000000
EMB_DIM = 64
BATCH = 16384

_NC = 2
_NS = 16
_LANES = 16
_NW = _NC * _NS
_BPW = BATCH // _NW
_CHUNK = 128
_NCHUNK = _BPW // _CHUNK
_NBUF = 8


def _row_scalar(idx_v, r):
    v = idx_v[lax.shift_right_logical(r, 7),
              pl.ds(lax.bitwise_and(r, 0x70), _LANES)]
    lane = lax.bitwise_and(r, 0xF)
    splat = lax.broadcast(lane, (_LANES,))
    vs = v[splat]
    return vs[0], vs


def _emb_mul_kernel(idx_hbm, rep_hbm, tabt_hbm, out_hbm,
                    idx_v, ring_v, rep_v, ob_v, gsem, rsem):
    wid = lax.axis_index("s") * _NC + lax.axis_index("c")
    base = pl.multiple_of(wid * _BPW, _BPW)

    pltpu.sync_copy(idx_hbm.at[pl.ds(wid * (_BPW // 128), _BPW // 128)],
                    idx_v)

    def fetch(r, slot):
        scalar, _ = _row_scalar(idx_v, r)
        col = pl.multiple_of(
            lax.shift_left(lax.shift_right_logical(scalar, 7), 7), 128)
        pltpu.async_copy(tabt_hbm.at[:, pl.ds(col, 128)],
                         ring_v.at[slot], gsem)

    def slot_wait():
        pltpu.make_async_copy(
            tabt_hbm.at[:, pl.ds(0, 128)], ring_v.at[0], gsem).wait()

    for s in range(_NBUF):
        fetch(jnp.int32(s), s)

    for j in range(_NCHUNK):
        rbase = base + j * _CHUNK
        rep_cp = pltpu.async_copy(rep_hbm.at[pl.ds(rbase, _CHUNK)],
                                  rep_v, rsem)
        rep_cp.wait()

        def group_body(g, carry):
            for s in range(_NBUF):
                r = j * _CHUNK + g * _NBUF + s
                rloc = g * _NBUF + s
                scalar, splat = _row_scalar(idx_v, r)
                lane_splat = lax.bitwise_and(splat, 127)
                slot_wait()
                nxt = r + _NBUF
                @pl.when(nxt < _BPW)
                def _():
                    fetch(nxt, s)
                slot_idx = jnp.full((_LANES,), s, jnp.int32)
                for c in range(EMB_DIM // _LANES):
                    jidx = lax.iota(jnp.int32, _LANES) + c * _LANES
                    g16 = plsc.load_gather(
                        ring_v, [slot_idx, jidx, lane_splat])
                    sl = pl.ds(c * _LANES, _LANES)
                    ob_v[rloc, sl] = g16 * rep_v[rloc, sl]
            return carry

        lax.fori_loop(0, _CHUNK // _NBUF, group_body, 0)

        pltpu.sync_copy(ob_v, out_hbm.at[pl.ds(rbase, _CHUNK)])


@jax.jit
def kernel(_next_types, representation, table):
    idx = _next_types.reshape(BATCH // 128, 128).astype(jnp.int32)
    rep = representation.reshape(BATCH, EMB_DIM)
    tabt = table.T

    mesh = plsc.VectorSubcoreMesh(core_axis_name="c", subcore_axis_name="s")
    out = pl.kernel(
        _emb_mul_kernel,
        out_type=jax.ShapeDtypeStruct((BATCH, EMB_DIM), jnp.float32),
        mesh=mesh,
        compiler_params=pltpu.CompilerParams(
            use_tc_tiling_on_sc=True, needs_layout_passes=False),
        scratch_types=[
            pltpu.VMEM((_BPW // 128, 128), jnp.int32),
            pltpu.VMEM((_NBUF, EMB_DIM, 128), jnp.float32),
            pltpu.VMEM((_CHUNK, EMB_DIM), jnp.float32),
            pltpu.VMEM((_CHUNK, EMB_DIM), jnp.float32),
            pltpu.SemaphoreType.DMA,
            pltpu.SemaphoreType.DMA,
        ],
    )(idx, rep, tabt)
    return out.reshape(BATCH, 1, EMB_DIM)


# speedup vs baseline: 5.0211x; 1.6567x over previous
import jax
import jax.numpy as jnp
from jax import lax
from jax.experimental import pallas as pl
from jax.experimental.pallas import tpu as pltpu
from jax.experimental.pallas import tpu_sc as plsc

VOCAB = 1000000
EMB_DIM = 64
BATCH = 16384

_NC = 2
_NS = 16
_LANES = 16
_NW = _NC * _NS
_BPW = BATCH // _NW
_CHUNK = 128
_NCHUNK = _BPW // _CHUNK
_NBUF = 8
_IOTA = lambda: lax.iota(jnp.int32, _LANES)


def _vec_at(arr_v, r):
    return arr_v[lax.shift_right_logical(r, 7),
                 pl.ds(lax.bitwise_and(r, 0x70), _LANES)]


def _splat_at(arr_v, r):
    v = _vec_at(arr_v, r)
    return v[lax.broadcast(lax.bitwise_and(r, 0xF), (_LANES,))]


def _scalar_at(arr_v, r):
    return _splat_at(arr_v, r)[0]


def _emb_mul_kernel(sidx_hbm, ord_hbm, rep_hbm, tabt_hbm, out_hbm,
                    sidx_v, ord_v, slots_v, flags_v, dbuck_v, cbuf_v,
                    ring_v, rep_v, ob_v, gsem, rsem, osem):
    wid = lax.axis_index("s") * _NC + lax.axis_index("c")

    pltpu.sync_copy(sidx_hbm.at[pl.ds(wid * (_BPW // 128), _BPW // 128)],
                    sidx_v)
    pltpu.sync_copy(ord_hbm.at[pl.ds(wid * (_BPW // 128), _BPW // 128)],
                    ord_v)

    cbuf_v[pl.ds(0, _LANES)] = lax.broadcast(jnp.int32(-1), (_LANES,))
    for t in range(_BPW // _LANES):
        j, off = t // 8, (t % 8) * _LANES
        c = lax.shift_right_logical(sidx_v[j, pl.ds(off, _LANES)], 7)
        cbuf_v[pl.ds(1 + t * _LANES, _LANES)] = c
    run = jnp.int32(0)
    for t in range(_BPW // _LANES):
        j, off = t // 8, (t % 8) * _LANES
        sl = pl.ds(off, _LANES)
        c = cbuf_v[pl.ds(1 + t * _LANES, _LANES)]
        pcs = cbuf_v[pl.ds(t * _LANES, _LANES)]
        flag = (c != pcs).astype(jnp.int32)
        cum = plsc.cumsum(flag)
        slots_v[j, sl] = lax.broadcast(run, (_LANES,)) + cum - 1
        flags_v[j, sl] = flag
        plsc.store_compressed(dbuck_v.at[pl.ds(run, _LANES)], c,
                              mask=flag == 1)
        run = run + cum[_LANES - 1]
    ndist = run

    def fetch(d, slot):
        v = dbuck_v[pl.ds(lax.bitwise_and(d, ~0xF), _LANES)]
        bucket = v[lax.broadcast(lax.bitwise_and(d, 0xF), (_LANES,))][0]
        col = pl.multiple_of(lax.shift_left(bucket, 7), 128)
        pltpu.async_copy(tabt_hbm.at[:, pl.ds(col, 128)],
                         ring_v.at[slot], gsem)

    def slot_wait():
        pltpu.make_async_copy(
            tabt_hbm.at[:, pl.ds(0, 128)], ring_v.at[0], gsem).wait()

    for s in range(_NBUF):
        @pl.when(s < ndist)
        def _():
            fetch(jnp.int32(s), s)

    for j in range(_NCHUNK):
        def rep_fire(r2, carry):
            ob = _scalar_at(ord_v, j * _CHUNK + r2)
            pltpu.async_copy(rep_hbm.at[pl.ds(ob, 1)],
                             rep_v.at[pl.ds(r2, 1)], rsem)
            return carry
        lax.fori_loop(0, _CHUNK, rep_fire, 0)
        pltpu.make_async_copy(
            rep_hbm.at[pl.ds(0, _CHUNK)], rep_v, rsem).wait()

        def row_body(r2, carry):
            r = j * _CHUNK + r2
            slot = _scalar_at(slots_v, r)
            flag = _scalar_at(flags_v, r)

            @pl.when(flag == 1)
            def _():
                slot_wait()
                nd = slot - 1 + _NBUF
                @pl.when(jnp.logical_and(slot >= 1, nd < ndist))
                def _():
                    fetch(nd, lax.rem(slot - 1, _NBUF))

            lane_splat = lax.bitwise_and(_splat_at(sidx_v, r), 127)
            slot_splat = lax.broadcast(lax.rem(slot, _NBUF), (_LANES,))
            for c in range(EMB_DIM // _LANES):
                jidx = _IOTA() + c * _LANES
                g16 = plsc.load_gather(
                    ring_v, [slot_splat, jidx, lane_splat])
                sl = pl.ds(c * _LANES, _LANES)
                ob_v[r2, sl] = g16 * rep_v[r2, sl]
            return carry

        lax.fori_loop(0, _CHUNK, row_body, 0)

        def out_fire(r2, carry):
            ob = _scalar_at(ord_v, j * _CHUNK + r2)
            pltpu.async_copy(ob_v.at[pl.ds(r2, 1)],
                             out_hbm.at[pl.ds(ob, 1)], osem)
            return carry
        lax.fori_loop(0, _CHUNK, out_fire, 0)
        pltpu.make_async_copy(
            ob_v, out_hbm.at[pl.ds(0, _CHUNK)], osem).wait()


@jax.jit
def kernel(_next_types, representation, table):
    idx = _next_types.astype(jnp.int32)
    sidx, order = jax.lax.sort_key_val(
        idx, lax.iota(jnp.int32, BATCH))
    sidx = sidx.reshape(BATCH // 128, 128)
    order = order.reshape(BATCH // 128, 128)
    rep = representation.reshape(BATCH, EMB_DIM)
    tabt = table.T

    mesh = plsc.VectorSubcoreMesh(core_axis_name="c", subcore_axis_name="s")
    out = pl.kernel(
        _emb_mul_kernel,
        out_type=jax.ShapeDtypeStruct((BATCH, EMB_DIM), jnp.float32),
        mesh=mesh,
        compiler_params=pltpu.CompilerParams(
            use_tc_tiling_on_sc=True, needs_layout_passes=False),
        scratch_types=[
            pltpu.VMEM((_BPW // 128, 128), jnp.int32),
            pltpu.VMEM((_BPW // 128, 128), jnp.int32),
            pltpu.VMEM((_BPW // 128, 128), jnp.int32),
            pltpu.VMEM((_BPW // 128, 128), jnp.int32),
            pltpu.VMEM((_BPW + _LANES,), jnp.int32),
            pltpu.VMEM((_BPW + 2 * _LANES,), jnp.int32),
            pltpu.VMEM((_NBUF, EMB_DIM, 128), jnp.float32),
            pltpu.VMEM((_CHUNK, EMB_DIM), jnp.float32),
            pltpu.VMEM((_CHUNK, EMB_DIM), jnp.float32),
            pltpu.SemaphoreType.DMA,
            pltpu.SemaphoreType.DMA,
            pltpu.SemaphoreType.DMA,
        ],
    )(sidx, order, rep, tabt)
    return out.reshape(BATCH, 1, EMB_DIM)


# speedup vs baseline: 5.0504x; 1.0058x over previous
import jax
import jax.numpy as jnp
from jax import lax
from jax.experimental import pallas as pl
from jax.experimental.pallas import tpu as pltpu
from jax.experimental.pallas import tpu_sc as plsc

VOCAB = 1000000
EMB_DIM = 64
BATCH = 16384

_NC = 2
_NS = 16
_LANES = 16
_NW = _NC * _NS
_BPW = BATCH // _NW
_CHUNK = 128
_NCHUNK = _BPW // _CHUNK
_NBUF = 6
_IOTA = lambda: lax.iota(jnp.int32, _LANES)


def _vec_at(arr_v, r):
    return arr_v[lax.shift_right_logical(r, 7),
                 pl.ds(lax.bitwise_and(r, 0x70), _LANES)]


def _splat_at(arr_v, r):
    v = _vec_at(arr_v, r)
    return v[lax.broadcast(lax.bitwise_and(r, 0xF), (_LANES,))]


def _scalar_at(arr_v, r):
    return _splat_at(arr_v, r)[0]


def _emb_mul_kernel(sidx_hbm, ord_hbm, rep_hbm, tabt_hbm, out_hbm,
                    sidx_v, ord_v, slots_v, flags_v, dbuck_v, cbuf_v,
                    ring_v, rep_v, ob_v, gsem, rsem, osem):
    wid = lax.axis_index("s") * _NC + lax.axis_index("c")

    pltpu.sync_copy(sidx_hbm.at[pl.ds(wid * (_BPW // 128), _BPW // 128)],
                    sidx_v)
    pltpu.sync_copy(ord_hbm.at[pl.ds(wid * (_BPW // 128), _BPW // 128)],
                    ord_v)

    cbuf_v[pl.ds(0, _LANES)] = lax.broadcast(jnp.int32(-1), (_LANES,))
    for t in range(_BPW // _LANES):
        j, off = t // 8, (t % 8) * _LANES
        c = lax.shift_right_logical(sidx_v[j, pl.ds(off, _LANES)], 7)
        cbuf_v[pl.ds(1 + t * _LANES, _LANES)] = c
    run = jnp.int32(0)
    for t in range(_BPW // _LANES):
        j, off = t // 8, (t % 8) * _LANES
        sl = pl.ds(off, _LANES)
        c = cbuf_v[pl.ds(1 + t * _LANES, _LANES)]
        pcs = cbuf_v[pl.ds(t * _LANES, _LANES)]
        flag = (c != pcs).astype(jnp.int32)
        cum = plsc.cumsum(flag)
        slots_v[j, sl] = lax.broadcast(run, (_LANES,)) + cum - 1
        flags_v[j, sl] = flag
        plsc.store_compressed(dbuck_v.at[pl.ds(run, _LANES)], c,
                              mask=flag == 1)
        run = run + cum[_LANES - 1]
    ndist = run

    def fetch(d, slot):
        v = dbuck_v[pl.ds(lax.bitwise_and(d, ~0xF), _LANES)]
        bucket = v[lax.broadcast(lax.bitwise_and(d, 0xF), (_LANES,))][0]
        col = pl.multiple_of(lax.shift_left(bucket, 7), 128)
        pltpu.async_copy(tabt_hbm.at[:, pl.ds(col, 128)],
                         ring_v.at[slot], gsem)

    def slot_wait():
        pltpu.make_async_copy(
            tabt_hbm.at[:, pl.ds(0, 128)], ring_v.at[0], gsem).wait()

    for s in range(_NBUF):
        @pl.when(s < ndist)
        def _():
            fetch(jnp.int32(s), s)

    def rep_fire_chunk(j, slot):
        def rep_fire(r2, carry):
            ob = _scalar_at(ord_v, j * _CHUNK + r2)
            pltpu.async_copy(rep_hbm.at[pl.ds(ob, 1)],
                             rep_v.at[slot, pl.ds(r2, 1)], rsem)
            return carry
        lax.fori_loop(0, _CHUNK, rep_fire, 0)

    rep_fire_chunk(0, 0)

    for j in range(_NCHUNK):
        pj = j % 2
        if j + 1 < _NCHUNK:
            rep_fire_chunk(j + 1, (j + 1) % 2)
        pltpu.make_async_copy(
            rep_hbm.at[pl.ds(0, _CHUNK)], rep_v.at[pj], rsem).wait()
        if j >= 2:
            pltpu.make_async_copy(
                ob_v.at[pj], out_hbm.at[pl.ds(0, _CHUNK)], osem).wait()

        def row_body(r2, carry):
            r = j * _CHUNK + r2
            slot = _scalar_at(slots_v, r)
            flag = _scalar_at(flags_v, r)

            @pl.when(flag == 1)
            def _():
                slot_wait()
                nd = slot - 1 + _NBUF
                @pl.when(jnp.logical_and(slot >= 1, nd < ndist))
                def _():
                    fetch(nd, lax.rem(slot - 1, _NBUF))

            lane_splat = lax.bitwise_and(_splat_at(sidx_v, r), 127)
            slot_splat = lax.broadcast(lax.rem(slot, _NBUF), (_LANES,))
            for c in range(EMB_DIM // _LANES):
                jidx = _IOTA() + c * _LANES
                g16 = plsc.load_gather(
                    ring_v, [slot_splat, jidx, lane_splat])
                sl = pl.ds(c * _LANES, _LANES)
                ob_v[pj, r2, sl] = g16 * rep_v[pj, r2, sl]
            return carry

        lax.fori_loop(0, _CHUNK, row_body, 0)

        def out_fire(r2, carry):
            ob = _scalar_at(ord_v, j * _CHUNK + r2)
            pltpu.async_copy(ob_v.at[pj, pl.ds(r2, 1)],
                             out_hbm.at[pl.ds(ob, 1)], osem)
            return carry
        lax.fori_loop(0, _CHUNK, out_fire, 0)

    for _ in range(min(2, _NCHUNK)):
        pltpu.make_async_copy(
            ob_v.at[0], out_hbm.at[pl.ds(0, _CHUNK)], osem).wait()


@jax.jit
def kernel(_next_types, representation, table):
    idx = _next_types.astype(jnp.int32)
    sidx, order = jax.lax.sort_key_val(
        idx, lax.iota(jnp.int32, BATCH))
    sidx = sidx.reshape(BATCH // 128, 128)
    order = order.reshape(BATCH // 128, 128)
    rep = representation.reshape(BATCH, EMB_DIM)
    tabt = table.T

    mesh = plsc.VectorSubcoreMesh(core_axis_name="c", subcore_axis_name="s")
    out = pl.kernel(
        _emb_mul_kernel,
        out_type=jax.ShapeDtypeStruct((BATCH, EMB_DIM), jnp.float32),
        mesh=mesh,
        compiler_params=pltpu.CompilerParams(
            use_tc_tiling_on_sc=True, needs_layout_passes=False),
        scratch_types=[
            pltpu.VMEM((_BPW // 128, 128), jnp.int32),
            pltpu.VMEM((_BPW // 128, 128), jnp.int32),
            pltpu.VMEM((_BPW // 128, 128), jnp.int32),
            pltpu.VMEM((_BPW // 128, 128), jnp.int32),
            pltpu.VMEM((_BPW + _LANES,), jnp.int32),
            pltpu.VMEM((_BPW + 2 * _LANES,), jnp.int32),
            pltpu.VMEM((_NBUF, EMB_DIM, 128), jnp.float32),
            pltpu.VMEM((2, _CHUNK, EMB_DIM), jnp.float32),
            pltpu.VMEM((2, _CHUNK, EMB_DIM), jnp.float32),
            pltpu.SemaphoreType.DMA,
            pltpu.SemaphoreType.DMA,
            pltpu.SemaphoreType.DMA,
        ],
    )(sidx, order, rep, tabt)
    return out.reshape(BATCH, 1, EMB_DIM)
